# baseline (speedup 1.0000x reference)
import functools
import jax, jax.numpy as jnp
from jax import lax
from jax.experimental import pallas as pl
from jax.experimental.pallas import tpu as pltpu
from jax.experimental.pallas import tpu_sc as plsc


def _sc_copy_body(x_hbm, o_hbm, stage, sem):
    c = lax.axis_index("c")
    s = lax.axis_index("s")
    wid = s * 2 + c
    n = x_hbm.shape[0]
    per = n // 32
    chunk = stage.shape[0] // 2
    base = wid * per

    def body(i, _):
        off = base + i * chunk
        pltpu.sync_copy(x_hbm.at[pl.ds(off, chunk)], stage.at[pl.ds(0, chunk)])
        pltpu.sync_copy(stage.at[pl.ds(0, chunk)], o_hbm.at[pl.ds(off, chunk)])
        return 0

    lax.fori_loop(0, per // chunk, body, 0)


def kernel(patches, masked_indices):
    B, N, P = patches.shape
    total = B * N * P
    mesh = plsc.VectorSubcoreMesh(core_axis_name="c", subcore_axis_name="s")
    out = functools.partial(
        pl.kernel,
        mesh=mesh,
        out_type=jax.ShapeDtypeStruct((total,), jnp.float32),
        scratch_types=[
            pltpu.VMEM((2 * 65536,), jnp.float32),
            pltpu.SemaphoreType.DMA,
        ],
    )(_sc_copy_body)(patches.reshape(total))
    return out.reshape(B, N, P)


# speedup vs baseline: 1.2628x; 1.2628x over previous
import functools

import jax
import jax.numpy as jnp
from jax import lax
from jax.experimental import pallas as pl


def _body(idx_ref, p1_ref, p2_ref, rm_ref, x_ref, o_ref, *, bb, p):
    p1 = p1_ref[...]
    p2 = p2_ref[...]
    rm = rm_ref[...]
    qi = lax.broadcasted_iota(jnp.int32, (64, 1), 0)
    li = lax.broadcasted_iota(jnp.int32, (1, rm.shape[1]), 1)
    t = (li % p).astype(jnp.float32) / (p - 1)
    for b in range(bb):
        idxr = idx_ref[b]
        hi = idxr // 64
        lo = idxr % 64
        a_t = (qi == hi).astype(jnp.float32)
        b2 = (qi == lo).astype(jnp.float32)
        cnt = lax.dot_general(a_t, b2, (((1,), (1,)), ((), ())),
                              preferred_element_type=jnp.float32)
        x = x_ref[b]
        starts = jnp.dot(x, p1, preferred_element_type=jnp.float32)
        ends = jnp.dot(x, p2, preferred_element_type=jnp.float32)
        s_exp = jnp.dot(starts, rm, preferred_element_type=jnp.float32)
        e_exp = jnp.dot(ends, rm, preferred_element_type=jnp.float32)
        lerp = s_exp + (e_exp - s_exp) * t
        mline = jnp.dot(cnt, rm, preferred_element_type=jnp.float32)
        o_ref[b] = jnp.where(mline > 0.0, lerp, x)


def kernel(patches, masked_indices):
    B, N, P = patches.shape
    M = masked_indices.shape[1]
    G = N // P
    W = N // G
    L = G * P
    idx3 = masked_indices.astype(jnp.int32).reshape(B, 1, M)

    li = jnp.arange(L, dtype=jnp.int32)[None, :]
    si = jnp.arange(W, dtype=jnp.int32)[:, None]
    sel = (li // P == si).astype(jnp.float32)
    t = (li % P).astype(jnp.float32) / (P - 1)
    p1 = jnp.transpose((li == si * P).astype(jnp.float32))
    p2 = jnp.transpose((li == si * P + (P - 1)).astype(jnp.float32))

    BB = 4
    cspec = lambda shp: pl.BlockSpec(shp, lambda b: (0,) * len(shp))
    out = pl.pallas_call(
        functools.partial(_body, bb=BB, p=P),
        grid=(B // BB,),
        in_specs=[
            pl.BlockSpec((BB, 1, M), lambda b: (b, 0, 0)),
            cspec((L, W)), cspec((L, W)), cspec((W, L)),
            pl.BlockSpec((BB, G, L), lambda b: (b, 0, 0)),
        ],
        out_specs=pl.BlockSpec((BB, G, L), lambda b: (b, 0, 0)),
        out_shape=jax.ShapeDtypeStruct((B, G, L), patches.dtype),
    )(idx3, p1, p2, sel, patches.reshape(B, G, L))
    return out.reshape(B, N, P)
